# Initial kernel scaffold; baseline (speedup 1.0000x reference)
import jax
import jax.numpy as jnp
from jax.experimental import pallas as pl


def kernel(v, e, batch, W, b):
    raise NotImplementedError("write your pallas kernel here")


# speedup vs baseline: 11.6563x; 11.6563x over previous
import functools

import jax
import jax.numpy as jnp
from jax import lax
from jax.experimental import pallas as pl
from jax.experimental.pallas import tpu as pltpu
from jax.experimental.pallas import tpu_sc as plsc

N_NODES = 10000
N_EDGES = 160000
D_IN = 256
D_OUT = 256
N_GRAPHS = 128
DH = 128
NC = 2
NS = 16
NW = NC * NS
L = 16

EPW = N_EDGES // NW
FULL_VREGS = EPW // L
TAIL = EPW - FULL_VREGS * L
PAD_E = FULL_VREGS * L + L


def _deg_body(col_hbm, degp_hbm, idx_v, deg_v):
    c = lax.axis_index("c")
    s = lax.axis_index("s")
    w = s * NC + c
    zeros16 = jnp.zeros((L,), jnp.float32)
    ones16 = jnp.ones((L,), jnp.float32)

    def zero_step(i, _):
        deg_v[pl.ds(i * L, L)] = zeros16
        return 0

    lax.fori_loop(0, N_NODES // L, zero_step, 0)
    pltpu.sync_copy(col_hbm.at[pl.ds(w * EPW, PAD_E)], idx_v)

    def hist_step(j, _):
        idx = idx_v[pl.ds(j * L, L)]
        plsc.addupdate_scatter(deg_v, [idx], ones16)
        return 0

    lax.fori_loop(0, FULL_VREGS, hist_step, 0)
    tail_idx = idx_v[pl.ds(FULL_VREGS * L, L)]
    tail_mask = lax.iota(jnp.int32, L) < TAIL
    plsc.addupdate_scatter(deg_v, [tail_idx], ones16, mask=tail_mask)
    pltpu.sync_copy(deg_v, degp_hbm.at[pl.ds(w * N_NODES, N_NODES)])


def _deg_partials(col_padded):
    mesh = plsc.VectorSubcoreMesh(core_axis_name="c", subcore_axis_name="s",
                                  num_cores=NC, num_subcores=NS)
    return pl.kernel(
        _deg_body,
        out_type=jax.ShapeDtypeStruct((NW * N_NODES,), jnp.float32),
        mesh=mesh,
        scratch_types=[
            pltpu.VMEM((PAD_E,), jnp.int32),
            pltpu.VMEM((N_NODES,), jnp.float32),
        ],
        compiler_params=pltpu.CompilerParams(needs_layout_passes=False),
    )(col_padded)


RB = 1000
NRB = N_NODES // RB


def _scale_mm_body(v_ref, w_ref, degp_ref, xl_ref, xr_ref):
    x = jnp.dot(v_ref[...], w_ref[...], preferred_element_type=jnp.float32)
    deg = jnp.sum(degp_ref[0], axis=0) + 1.0
    dinv = lax.rsqrt(deg)
    xs = x * dinv[:, None]
    xl_ref[...] = xs[:, :DH]
    xr_ref[...] = xs[:, DH:]


def _scaled_x(v, W, degp):
    return pl.pallas_call(
        _scale_mm_body,
        grid=(NRB,),
        in_specs=[
            pl.BlockSpec((RB, D_IN), lambda i: (i, 0)),
            pl.BlockSpec((D_IN, D_OUT), lambda i: (0, 0)),
            pl.BlockSpec((1, NW, RB), lambda i: (i, 0, 0)),
        ],
        out_specs=[
            pl.BlockSpec((RB, DH), lambda i: (i, 0)),
            pl.BlockSpec((RB, DH), lambda i: (i, 0)),
        ],
        out_shape=[
            jax.ShapeDtypeStruct((N_NODES, DH), jnp.float32),
            jax.ShapeDtypeStruct((N_NODES, DH), jnp.float32),
        ],
    )(v, W, degp)


EPT = N_EDGES // NS
CK = 80
NCHUNK = EPT // CK
STRIPE = 640
N_PAD = STRIPE * NS
FR = 160


def _agg_body(xl_hbm, xr_hbm, row_hbm, col_hbm, outl_hbm, outr_hbm,
              idxr_v, idxc_v, rows_v, stripe_v, acc_sh, sem):
    c = lax.axis_index("c")
    s = lax.axis_index("s")
    zeros16 = jnp.zeros((L,), jnp.float32)

    def zb_step(i, _):
        for j in range(DH // L):
            stripe_v[i, pl.ds(j * L, L)] = zeros16
        return 0

    lax.fori_loop(0, FR, zb_step, 0)
    sbase = pl.multiple_of(s * STRIPE, 8)

    def zs_step(i, _):
        pltpu.sync_copy(stripe_v, acc_sh.at[pl.ds(sbase + i * FR, FR)])
        return 0

    lax.fori_loop(0, STRIPE // FR, zs_step, 0)
    plsc.subcore_barrier()

    def run(x_hbm):
        def chunk_step(j, _):
            base = pl.multiple_of(s * EPT + j * CK, 8)
            pltpu.sync_copy(row_hbm.at[pl.ds(base, CK)], idxr_v)
            pltpu.sync_copy(col_hbm.at[pl.ds(base, CK)], idxc_v)
            pltpu.async_copy(x_hbm.at[idxr_v], rows_v, sem).wait()
            pltpu.sync_copy(rows_v, acc_sh.at[idxc_v], add=True)
            return 0

        lax.fori_loop(0, NCHUNK, chunk_step, 0)

    def flush(out_hbm):
        def f_step(i, _):
            off = pl.multiple_of(sbase + i * FR, 8)
            pltpu.sync_copy(acc_sh.at[pl.ds(off, FR)], stripe_v)
            pltpu.sync_copy(stripe_v, out_hbm.at[pl.ds(off, FR)])
            return 0

        lax.fori_loop(0, STRIPE // FR, f_step, 0)

    @pl.when(c == 0)
    def _():
        run(xl_hbm)

    @pl.when(c == 1)
    def _():
        run(xr_hbm)

    plsc.subcore_barrier()

    @pl.when(c == 0)
    def _():
        flush(outl_hbm)

    @pl.when(c == 1)
    def _():
        flush(outr_hbm)


def _edge_agg(xl, xr, row, col):
    mesh = plsc.VectorSubcoreMesh(core_axis_name="c", subcore_axis_name="s",
                                  num_cores=NC, num_subcores=NS)
    return pl.kernel(
        _agg_body,
        out_type=(jax.ShapeDtypeStruct((N_PAD, DH), jnp.float32),
                  jax.ShapeDtypeStruct((N_PAD, DH), jnp.float32)),
        mesh=mesh,
        scratch_types=[
            pltpu.VMEM((CK,), jnp.int32),
            pltpu.VMEM((CK,), jnp.int32),
            pltpu.VMEM((CK, DH), jnp.float32),
            pltpu.VMEM((FR, DH), jnp.float32),
            pltpu.VMEM_SHARED((N_PAD, DH), jnp.float32),
            pltpu.SemaphoreType.DMA,
        ],
        compiler_params=pltpu.CompilerParams(needs_layout_passes=False),
    )(xl, xr, row, col)


def _pool_body(aggl_ref, aggr_ref, xl_ref, xr_ref, degp_ref, b_ref, batch_ref,
               out_ref, sum_acc, cnt_acc):
    i = pl.program_id(0)
    deg = jnp.sum(degp_ref[0], axis=0) + 1.0
    dinv = lax.rsqrt(deg)
    agg = jnp.concatenate([aggl_ref[...], aggr_ref[...]], axis=1)
    xs = jnp.concatenate([xl_ref[...], xr_ref[...]], axis=1)
    h = jnp.maximum(dinv[:, None] * (agg + xs) + b_ref[...], 0.0)
    bvec = batch_ref[...].reshape(1, RB)
    onehot = (lax.broadcasted_iota(jnp.int32, (N_GRAPHS, RB), 0)
              == bvec).astype(jnp.float32)

    @pl.when(i == 0)
    def _():
        sum_acc[...] = jnp.zeros_like(sum_acc)
        cnt_acc[...] = jnp.zeros_like(cnt_acc)

    sum_acc[...] += jnp.dot(onehot, h, preferred_element_type=jnp.float32)
    cnt_acc[...] += jnp.dot(onehot, jnp.ones((RB, D_OUT), jnp.float32),
                            preferred_element_type=jnp.float32)

    @pl.when(i == NRB - 1)
    def _():
        out_ref[...] = sum_acc[...] / jnp.maximum(cnt_acc[...], 1.0)


def _pool(aggl, aggr, xl, xr, degp, b2, batch3):
    return pl.pallas_call(
        _pool_body,
        grid=(NRB,),
        in_specs=[
            pl.BlockSpec((RB, DH), lambda i: (i, 0)),
            pl.BlockSpec((RB, DH), lambda i: (i, 0)),
            pl.BlockSpec((RB, DH), lambda i: (i, 0)),
            pl.BlockSpec((RB, DH), lambda i: (i, 0)),
            pl.BlockSpec((1, NW, RB), lambda i: (i, 0, 0)),
            pl.BlockSpec((1, D_OUT), lambda i: (0, 0)),
            pl.BlockSpec((1, 1, RB), lambda i: (i, 0, 0)),
        ],
        out_specs=pl.BlockSpec((N_GRAPHS, D_OUT), lambda i: (0, 0)),
        out_shape=jax.ShapeDtypeStruct((N_GRAPHS, D_OUT), jnp.float32),
        scratch_shapes=[
            pltpu.VMEM((N_GRAPHS, D_OUT), jnp.float32),
            pltpu.VMEM((N_GRAPHS, D_OUT), jnp.float32),
        ],
    )(aggl, aggr, xl, xr, degp, b2, batch3)


def kernel(v, e, batch, W, b):
    e = e.astype(jnp.int32)
    row, col = e[0], e[1]
    pad = NW * EPW + PAD_E - N_EDGES
    col_padded = jnp.concatenate([col, jnp.zeros((pad,), jnp.int32)])
    degp = _deg_partials(col_padded)
    degp_t = degp.reshape(NW, NRB, RB).transpose(1, 0, 2)
    xl, xr = _scaled_x(v, W, degp_t)
    aggl, aggr = _edge_agg(xl, xr, row, col)
    aggl, aggr = aggl[:N_NODES], aggr[:N_NODES]
    return _pool(aggl, aggr, xl, xr, degp_t, b.reshape(1, D_OUT),
                 batch.astype(jnp.int32).reshape(NRB, 1, RB))


# speedup vs baseline: 21.9139x; 1.8800x over previous
import functools

import jax
import jax.numpy as jnp
from jax import lax
from jax.experimental import pallas as pl
from jax.experimental.pallas import tpu as pltpu
from jax.experimental.pallas import tpu_sc as plsc

N_NODES = 10000
N_EDGES = 160000
D_IN = 256
D_OUT = 256
N_GRAPHS = 128
DH = 128
NC = 2
NS = 16
NW = NC * NS
L = 16

EPW = N_EDGES // NW
FULL_VREGS = EPW // L
TAIL = EPW - FULL_VREGS * L
PAD_E = FULL_VREGS * L + L


def _deg_body(col_hbm, degp_hbm, idx_v, deg_v):
    c = lax.axis_index("c")
    s = lax.axis_index("s")
    w = s * NC + c
    zeros16 = jnp.zeros((L,), jnp.float32)
    ones16 = jnp.ones((L,), jnp.float32)

    def zero_step(i, _):
        deg_v[pl.ds(i * L, L)] = zeros16
        return 0

    lax.fori_loop(0, N_NODES // L, zero_step, 0)
    pltpu.sync_copy(col_hbm.at[pl.ds(w * EPW, PAD_E)], idx_v)

    def hist_step(j, _):
        idx = idx_v[pl.ds(j * L, L)]
        plsc.addupdate_scatter(deg_v, [idx], ones16)
        return 0

    lax.fori_loop(0, FULL_VREGS, hist_step, 0)
    tail_idx = idx_v[pl.ds(FULL_VREGS * L, L)]
    tail_mask = lax.iota(jnp.int32, L) < TAIL
    plsc.addupdate_scatter(deg_v, [tail_idx], ones16, mask=tail_mask)
    pltpu.sync_copy(deg_v, degp_hbm.at[pl.ds(w * N_NODES, N_NODES)])


def _deg_partials(col_padded):
    mesh = plsc.VectorSubcoreMesh(core_axis_name="c", subcore_axis_name="s",
                                  num_cores=NC, num_subcores=NS)
    return pl.kernel(
        _deg_body,
        out_type=jax.ShapeDtypeStruct((NW * N_NODES,), jnp.float32),
        mesh=mesh,
        scratch_types=[
            pltpu.VMEM((PAD_E,), jnp.int32),
            pltpu.VMEM((N_NODES,), jnp.float32),
        ],
        compiler_params=pltpu.CompilerParams(needs_layout_passes=False),
    )(col_padded)


RB = 1000
NRB = N_NODES // RB


def _scale_mm_body(v_ref, w_ref, degp_ref, xl_ref, xr_ref):
    x = jnp.dot(v_ref[...], w_ref[...], preferred_element_type=jnp.float32)
    deg = jnp.sum(degp_ref[0], axis=0) + 1.0
    dinv = lax.rsqrt(deg)
    xs = x * dinv[:, None]
    xl_ref[...] = xs[:, :DH]
    xr_ref[...] = xs[:, DH:]


def _scaled_x(v, W, degp):
    return pl.pallas_call(
        _scale_mm_body,
        grid=(NRB,),
        in_specs=[
            pl.BlockSpec((RB, D_IN), lambda i: (i, 0)),
            pl.BlockSpec((D_IN, D_OUT), lambda i: (0, 0)),
            pl.BlockSpec((1, NW, RB), lambda i: (i, 0, 0)),
        ],
        out_specs=[
            pl.BlockSpec((RB, DH), lambda i: (i, 0)),
            pl.BlockSpec((RB, DH), lambda i: (i, 0)),
        ],
        out_shape=[
            jax.ShapeDtypeStruct((N_NODES, DH), jnp.float32),
            jax.ShapeDtypeStruct((N_NODES, DH), jnp.float32),
        ],
    )(v, W, degp)


EPT = N_EDGES // NS
CK = 100
NCHUNK = EPT // CK
STRIPE = 640
N_PAD = STRIPE * NS
FR = 160


def _agg_body(xl_hbm, xr_hbm, e3_hbm, outl_hbm, outr_hbm,
              idx0_v, idx1_v, rows0_v, rows1_v, stripe_v, acc_sh,
              gsem0, gsem1):
    c = lax.axis_index("c")
    s = lax.axis_index("s")
    zeros16 = jnp.zeros((L,), jnp.float32)
    idx = (idx0_v, idx1_v)
    rows = (rows0_v, rows1_v)
    gsem = (gsem0, gsem1)

    def zb_step(i, _):
        for j in range(DH // L):
            stripe_v[i, pl.ds(j * L, L)] = zeros16
        return 0

    lax.fori_loop(0, FR, zb_step, 0)
    sbase = pl.multiple_of(s * STRIPE, 8)

    def zs_step(i, _):
        pltpu.sync_copy(stripe_v, acc_sh.at[pl.ds(sbase + i * FR, FR)])
        return 0

    lax.fori_loop(0, STRIPE // FR, zs_step, 0)
    plsc.subcore_barrier()

    def run(x_hbm):
        def start_gather(jn, q):
            pltpu.sync_copy(e3_hbm.at[s * NCHUNK + jn], idx[q])
            pltpu.async_copy(x_hbm.at[idx[q].at[0]], rows[q], gsem[q])

        start_gather(0, 0)

        def pair_step(g, _):
            for p in range(2):
                j = 2 * g + p
                q = 1 - p

                @pl.when(j + 1 < NCHUNK)
                def _():
                    start_gather(j + 1, q)

                pltpu.make_async_copy(x_hbm.at[idx[p].at[0]], rows[p],
                                      gsem[p]).wait()
                pltpu.sync_copy(rows[p], acc_sh.at[idx[p].at[1]], add=True)
            return 0

        lax.fori_loop(0, NCHUNK // 2, pair_step, 0)

    def flush(out_hbm):
        def f_step(i, _):
            off = pl.multiple_of(sbase + i * FR, 8)
            pltpu.sync_copy(acc_sh.at[pl.ds(off, FR)], stripe_v)
            pltpu.sync_copy(stripe_v, out_hbm.at[pl.ds(off, FR)])
            return 0

        lax.fori_loop(0, STRIPE // FR, f_step, 0)

    @pl.when(c == 0)
    def _():
        run(xl_hbm)

    @pl.when(c == 1)
    def _():
        run(xr_hbm)

    plsc.subcore_barrier()

    @pl.when(c == 0)
    def _():
        flush(outl_hbm)

    @pl.when(c == 1)
    def _():
        flush(outr_hbm)


def _edge_agg(xl, xr, e3):
    mesh = plsc.VectorSubcoreMesh(core_axis_name="c", subcore_axis_name="s",
                                  num_cores=NC, num_subcores=NS)
    return pl.kernel(
        _agg_body,
        out_type=(jax.ShapeDtypeStruct((N_PAD, DH), jnp.float32),
                  jax.ShapeDtypeStruct((N_PAD, DH), jnp.float32)),
        mesh=mesh,
        scratch_types=[
            pltpu.VMEM((2, CK), jnp.int32),
            pltpu.VMEM((2, CK), jnp.int32),
            pltpu.VMEM((CK, DH), jnp.float32),
            pltpu.VMEM((CK, DH), jnp.float32),
            pltpu.VMEM((FR, DH), jnp.float32),
            pltpu.VMEM_SHARED((N_PAD, DH), jnp.float32),
            pltpu.SemaphoreType.DMA,
            pltpu.SemaphoreType.DMA,
        ],
        compiler_params=pltpu.CompilerParams(needs_layout_passes=False),
    )(xl, xr, e3)


def _pool_body(aggl_ref, aggr_ref, xl_ref, xr_ref, degp_ref, b_ref, batch_ref,
               out_ref, sum_acc, cnt_acc):
    i = pl.program_id(0)
    deg = jnp.sum(degp_ref[0], axis=0) + 1.0
    dinv = lax.rsqrt(deg)
    agg = jnp.concatenate([aggl_ref[...], aggr_ref[...]], axis=1)
    xs = jnp.concatenate([xl_ref[...], xr_ref[...]], axis=1)
    h = jnp.maximum(dinv[:, None] * (agg + xs) + b_ref[...], 0.0)
    bvec = batch_ref[...].reshape(1, RB)
    onehot = (lax.broadcasted_iota(jnp.int32, (N_GRAPHS, RB), 0)
              == bvec).astype(jnp.float32)

    @pl.when(i == 0)
    def _():
        sum_acc[...] = jnp.zeros_like(sum_acc)
        cnt_acc[...] = jnp.zeros_like(cnt_acc)

    sum_acc[...] += jnp.dot(onehot, h, preferred_element_type=jnp.float32)
    cnt_acc[...] += jnp.dot(onehot, jnp.ones((RB, D_OUT), jnp.float32),
                            preferred_element_type=jnp.float32)

    @pl.when(i == NRB - 1)
    def _():
        out_ref[...] = sum_acc[...] / jnp.maximum(cnt_acc[...], 1.0)


def _pool(aggl, aggr, xl, xr, degp, b2, batch3):
    return pl.pallas_call(
        _pool_body,
        grid=(NRB,),
        in_specs=[
            pl.BlockSpec((RB, DH), lambda i: (i, 0)),
            pl.BlockSpec((RB, DH), lambda i: (i, 0)),
            pl.BlockSpec((RB, DH), lambda i: (i, 0)),
            pl.BlockSpec((RB, DH), lambda i: (i, 0)),
            pl.BlockSpec((1, NW, RB), lambda i: (i, 0, 0)),
            pl.BlockSpec((1, D_OUT), lambda i: (0, 0)),
            pl.BlockSpec((1, 1, RB), lambda i: (i, 0, 0)),
        ],
        out_specs=pl.BlockSpec((N_GRAPHS, D_OUT), lambda i: (0, 0)),
        out_shape=jax.ShapeDtypeStruct((N_GRAPHS, D_OUT), jnp.float32),
        scratch_shapes=[
            pltpu.VMEM((N_GRAPHS, D_OUT), jnp.float32),
            pltpu.VMEM((N_GRAPHS, D_OUT), jnp.float32),
        ],
    )(aggl, aggr, xl, xr, degp, b2, batch3)


def kernel(v, e, batch, W, b):
    e = e.astype(jnp.int32)
    row, col = e[0], e[1]
    pad = NW * EPW + PAD_E - N_EDGES
    col_padded = jnp.concatenate([col, jnp.zeros((pad,), jnp.int32)])
    degp = _deg_partials(col_padded)
    degp_t = degp.reshape(NW, NRB, RB).transpose(1, 0, 2)
    xl, xr = _scaled_x(v, W, degp_t)
    e3 = (e.reshape(2, NS, NCHUNK, CK).transpose(1, 2, 0, 3)
          .reshape(NS * NCHUNK, 2, CK))
    aggl, aggr = _edge_agg(xl, xr, e3)
    aggl, aggr = aggl[:N_NODES], aggr[:N_NODES]
    return _pool(aggl, aggr, xl, xr, degp_t, b.reshape(1, D_OUT),
                 batch.astype(jnp.int32).reshape(NRB, 1, RB))


# speedup vs baseline: 21.9395x; 1.0012x over previous
import functools

import jax
import jax.numpy as jnp
from jax import lax
from jax.experimental import pallas as pl
from jax.experimental.pallas import tpu as pltpu
from jax.experimental.pallas import tpu_sc as plsc

N_NODES = 10000
N_EDGES = 160000
D_IN = 256
D_OUT = 256
N_GRAPHS = 128
DH = 128
NC = 2
NS = 16
NW = NC * NS
L = 16

EPW = N_EDGES // NW
FULL_VREGS = EPW // L
TAIL = EPW - FULL_VREGS * L
PAD_E = FULL_VREGS * L + L


def _deg_body(col_hbm, degp_hbm, idx_v, deg_v):
    c = lax.axis_index("c")
    s = lax.axis_index("s")
    w = s * NC + c
    zeros16 = jnp.zeros((L,), jnp.float32)
    ones16 = jnp.ones((L,), jnp.float32)

    def zero_step(i, _):
        deg_v[pl.ds(i * L, L)] = zeros16
        return 0

    lax.fori_loop(0, N_NODES // L, zero_step, 0)
    pltpu.sync_copy(col_hbm.at[pl.ds(w * EPW, PAD_E)], idx_v)

    def hist_step(j, _):
        idx = idx_v[pl.ds(j * L, L)]
        plsc.addupdate_scatter(deg_v, [idx], ones16)
        return 0

    lax.fori_loop(0, FULL_VREGS, hist_step, 0)
    tail_idx = idx_v[pl.ds(FULL_VREGS * L, L)]
    tail_mask = lax.iota(jnp.int32, L) < TAIL
    plsc.addupdate_scatter(deg_v, [tail_idx], ones16, mask=tail_mask)
    pltpu.sync_copy(deg_v, degp_hbm.at[pl.ds(w * N_NODES, N_NODES)])


def _deg_partials(col_padded):
    mesh = plsc.VectorSubcoreMesh(core_axis_name="c", subcore_axis_name="s",
                                  num_cores=NC, num_subcores=NS)
    return pl.kernel(
        _deg_body,
        out_type=jax.ShapeDtypeStruct((NW * N_NODES,), jnp.float32),
        mesh=mesh,
        scratch_types=[
            pltpu.VMEM((PAD_E,), jnp.int32),
            pltpu.VMEM((N_NODES,), jnp.float32),
        ],
        compiler_params=pltpu.CompilerParams(needs_layout_passes=False),
    )(col_padded)


RB = 1000
NRB = N_NODES // RB


def _scale_mm_body(v_ref, w_ref, degp_ref, xl_ref, xr_ref):
    x = jnp.dot(v_ref[...], w_ref[...], preferred_element_type=jnp.float32)
    deg = jnp.sum(degp_ref[0], axis=0) + 1.0
    dinv = lax.rsqrt(deg)
    xs = x * dinv[:, None]
    xl_ref[...] = xs[:, :DH]
    xr_ref[...] = xs[:, DH:]


def _scaled_x(v, W, degp):
    return pl.pallas_call(
        _scale_mm_body,
        grid=(NRB,),
        in_specs=[
            pl.BlockSpec((RB, D_IN), lambda i: (i, 0)),
            pl.BlockSpec((D_IN, D_OUT), lambda i: (0, 0)),
            pl.BlockSpec((1, NW, RB), lambda i: (i, 0, 0)),
        ],
        out_specs=[
            pl.BlockSpec((RB, DH), lambda i: (i, 0)),
            pl.BlockSpec((RB, DH), lambda i: (i, 0)),
        ],
        out_shape=[
            jax.ShapeDtypeStruct((N_NODES, DH), jnp.float32),
            jax.ShapeDtypeStruct((N_NODES, DH), jnp.float32),
        ],
    )(v, W, degp)


EPT = N_EDGES // NS
CK = 100
NCHUNK = EPT // CK
STRIPE = 640
N_PAD = STRIPE * NS
FR = 160


def _agg_body(xl_hbm, xr_hbm, e3_hbm, outl_hbm, outr_hbm,
              idx0_v, idx1_v, rows0_v, rows1_v, stripe_v, acc_sh,
              gsem0, gsem1, ssem0, ssem1):
    c = lax.axis_index("c")
    s = lax.axis_index("s")
    zeros16 = jnp.zeros((L,), jnp.float32)
    idx = (idx0_v, idx1_v)
    rows = (rows0_v, rows1_v)
    gsem = (gsem0, gsem1)
    ssem = (ssem0, ssem1)

    def zb_step(i, _):
        for j in range(DH // L):
            stripe_v[i, pl.ds(j * L, L)] = zeros16
        return 0

    lax.fori_loop(0, FR, zb_step, 0)
    sbase = pl.multiple_of(s * STRIPE, 8)

    def zs_step(i, _):
        pltpu.sync_copy(stripe_v, acc_sh.at[pl.ds(sbase + i * FR, FR)])
        return 0

    lax.fori_loop(0, STRIPE // FR, zs_step, 0)
    plsc.subcore_barrier()

    def run(x_hbm):
        def start_gather(jn, q):
            pltpu.sync_copy(e3_hbm.at[s * NCHUNK + jn], idx[q])
            pltpu.async_copy(x_hbm.at[idx[q].at[0]], rows[q], gsem[q])

        start_gather(0, 0)

        def pair_step(g, _):
            for p in range(2):
                j = 2 * g + p
                q = 1 - p

                @pl.when(j + 1 < NCHUNK)
                def _():
                    @pl.when(j > 0)
                    def _():
                        pltpu.make_async_copy(
                            rows[q], acc_sh.at[idx[q].at[1]], ssem[q]).wait()

                    start_gather(j + 1, q)

                pltpu.make_async_copy(x_hbm.at[idx[p].at[0]], rows[p],
                                      gsem[p]).wait()
                pltpu.async_copy(rows[p], acc_sh.at[idx[p].at[1]], ssem[p],
                                 add=True)
            return 0

        lax.fori_loop(0, NCHUNK // 2, pair_step, 0)
        pltpu.make_async_copy(rows[0], acc_sh.at[idx[0].at[1]], ssem[0]).wait()
        pltpu.make_async_copy(rows[1], acc_sh.at[idx[1].at[1]], ssem[1]).wait()

    def flush(out_hbm):
        def f_step(i, _):
            off = pl.multiple_of(sbase + i * FR, 8)
            pltpu.sync_copy(acc_sh.at[pl.ds(off, FR)], stripe_v)
            pltpu.sync_copy(stripe_v, out_hbm.at[pl.ds(off, FR)])
            return 0

        lax.fori_loop(0, STRIPE // FR, f_step, 0)

    @pl.when(c == 0)
    def _():
        run(xl_hbm)

    @pl.when(c == 1)
    def _():
        run(xr_hbm)

    plsc.subcore_barrier()

    @pl.when(c == 0)
    def _():
        flush(outl_hbm)

    @pl.when(c == 1)
    def _():
        flush(outr_hbm)


def _edge_agg(xl, xr, e3):
    mesh = plsc.VectorSubcoreMesh(core_axis_name="c", subcore_axis_name="s",
                                  num_cores=NC, num_subcores=NS)
    return pl.kernel(
        _agg_body,
        out_type=(jax.ShapeDtypeStruct((N_PAD, DH), jnp.float32),
                  jax.ShapeDtypeStruct((N_PAD, DH), jnp.float32)),
        mesh=mesh,
        scratch_types=[
            pltpu.VMEM((2, CK), jnp.int32),
            pltpu.VMEM((2, CK), jnp.int32),
            pltpu.VMEM((CK, DH), jnp.float32),
            pltpu.VMEM((CK, DH), jnp.float32),
            pltpu.VMEM((FR, DH), jnp.float32),
            pltpu.VMEM_SHARED((N_PAD, DH), jnp.float32),
            pltpu.SemaphoreType.DMA,
            pltpu.SemaphoreType.DMA,
            pltpu.SemaphoreType.DMA,
            pltpu.SemaphoreType.DMA,
        ],
        compiler_params=pltpu.CompilerParams(needs_layout_passes=False),
    )(xl, xr, e3)


def _pool_body(aggl_ref, aggr_ref, xl_ref, xr_ref, degp_ref, b_ref, batch_ref,
               out_ref, sum_acc, cnt_acc):
    i = pl.program_id(0)
    deg = jnp.sum(degp_ref[0], axis=0) + 1.0
    dinv = lax.rsqrt(deg)
    agg = jnp.concatenate([aggl_ref[...], aggr_ref[...]], axis=1)
    xs = jnp.concatenate([xl_ref[...], xr_ref[...]], axis=1)
    h = jnp.maximum(dinv[:, None] * (agg + xs) + b_ref[...], 0.0)
    bvec = batch_ref[...].reshape(1, RB)
    onehot = (lax.broadcasted_iota(jnp.int32, (N_GRAPHS, RB), 0)
              == bvec).astype(jnp.float32)

    @pl.when(i == 0)
    def _():
        sum_acc[...] = jnp.zeros_like(sum_acc)
        cnt_acc[...] = jnp.zeros_like(cnt_acc)

    sum_acc[...] += jnp.dot(onehot, h, preferred_element_type=jnp.float32)
    cnt_acc[...] += jnp.dot(onehot, jnp.ones((RB, D_OUT), jnp.float32),
                            preferred_element_type=jnp.float32)

    @pl.when(i == NRB - 1)
    def _():
        out_ref[...] = sum_acc[...] / jnp.maximum(cnt_acc[...], 1.0)


def _pool(aggl, aggr, xl, xr, degp, b2, batch3):
    return pl.pallas_call(
        _pool_body,
        grid=(NRB,),
        in_specs=[
            pl.BlockSpec((RB, DH), lambda i: (i, 0)),
            pl.BlockSpec((RB, DH), lambda i: (i, 0)),
            pl.BlockSpec((RB, DH), lambda i: (i, 0)),
            pl.BlockSpec((RB, DH), lambda i: (i, 0)),
            pl.BlockSpec((1, NW, RB), lambda i: (i, 0, 0)),
            pl.BlockSpec((1, D_OUT), lambda i: (0, 0)),
            pl.BlockSpec((1, 1, RB), lambda i: (i, 0, 0)),
        ],
        out_specs=pl.BlockSpec((N_GRAPHS, D_OUT), lambda i: (0, 0)),
        out_shape=jax.ShapeDtypeStruct((N_GRAPHS, D_OUT), jnp.float32),
        scratch_shapes=[
            pltpu.VMEM((N_GRAPHS, D_OUT), jnp.float32),
            pltpu.VMEM((N_GRAPHS, D_OUT), jnp.float32),
        ],
    )(aggl, aggr, xl, xr, degp, b2, batch3)


def kernel(v, e, batch, W, b):
    e = e.astype(jnp.int32)
    row, col = e[0], e[1]
    pad = NW * EPW + PAD_E - N_EDGES
    col_padded = jnp.concatenate([col, jnp.zeros((pad,), jnp.int32)])
    degp = _deg_partials(col_padded)
    degp_t = degp.reshape(NW, NRB, RB).transpose(1, 0, 2)
    xl, xr = _scaled_x(v, W, degp_t)
    e3 = (e.reshape(2, NS, NCHUNK, CK).transpose(1, 2, 0, 3)
          .reshape(NS * NCHUNK, 2, CK))
    aggl, aggr = _edge_agg(xl, xr, e3)
    aggl, aggr = aggl[:N_NODES], aggr[:N_NODES]
    return _pool(aggl, aggr, xl, xr, degp_t, b.reshape(1, D_OUT),
                 batch.astype(jnp.int32).reshape(NRB, 1, RB))


# speedup vs baseline: 24.8289x; 1.1317x over previous
import functools

import jax
import jax.numpy as jnp
from jax import lax
from jax.experimental import pallas as pl
from jax.experimental.pallas import tpu as pltpu
from jax.experimental.pallas import tpu_sc as plsc

N_NODES = 10000
N_EDGES = 160000
D_IN = 256
D_OUT = 256
N_GRAPHS = 128
DH = 128
NC = 2
NS = 16
NW = NC * NS
L = 16

EPW = N_EDGES // NW
FULL_VREGS = EPW // L
TAIL = EPW - FULL_VREGS * L
PAD_E = FULL_VREGS * L + L


def _deg_body(col_hbm, degp_hbm, idx_v, deg_v):
    c = lax.axis_index("c")
    s = lax.axis_index("s")
    w = s * NC + c
    zeros16 = jnp.zeros((L,), jnp.float32)
    ones16 = jnp.ones((L,), jnp.float32)

    def zero_step(i, _):
        deg_v[pl.ds(i * L, L)] = zeros16
        return 0

    lax.fori_loop(0, N_NODES // L, zero_step, 0)
    pltpu.sync_copy(col_hbm.at[pl.ds(w * EPW, PAD_E)], idx_v)

    def hist_step(j, _):
        idx = idx_v[pl.ds(j * L, L)]
        plsc.addupdate_scatter(deg_v, [idx], ones16)
        return 0

    lax.fori_loop(0, FULL_VREGS, hist_step, 0)
    tail_idx = idx_v[pl.ds(FULL_VREGS * L, L)]
    tail_mask = lax.iota(jnp.int32, L) < TAIL
    plsc.addupdate_scatter(deg_v, [tail_idx], ones16, mask=tail_mask)
    pltpu.sync_copy(deg_v, degp_hbm.at[pl.ds(w * N_NODES, N_NODES)])


def _deg_partials(col_padded):
    mesh = plsc.VectorSubcoreMesh(core_axis_name="c", subcore_axis_name="s",
                                  num_cores=NC, num_subcores=NS)
    return pl.kernel(
        _deg_body,
        out_type=jax.ShapeDtypeStruct((NW * N_NODES,), jnp.float32),
        mesh=mesh,
        scratch_types=[
            pltpu.VMEM((PAD_E,), jnp.int32),
            pltpu.VMEM((N_NODES,), jnp.float32),
        ],
        compiler_params=pltpu.CompilerParams(needs_layout_passes=False),
    )(col_padded)


RB = 1000
NRB = N_NODES // RB


def _scale_mm_body(v_ref, w_ref, degp_ref, xl_ref, xr_ref):
    x = jnp.dot(v_ref[...], w_ref[...], preferred_element_type=jnp.float32)
    deg = jnp.sum(degp_ref[0], axis=0) + 1.0
    dinv = lax.rsqrt(deg)
    xs = x * dinv[:, None]
    xl_ref[...] = xs[:, :DH]
    xr_ref[...] = xs[:, DH:]


def _scaled_x(v, W, degp):
    return pl.pallas_call(
        _scale_mm_body,
        grid=(NRB,),
        in_specs=[
            pl.BlockSpec((RB, D_IN), lambda i: (i, 0)),
            pl.BlockSpec((D_IN, D_OUT), lambda i: (0, 0)),
            pl.BlockSpec((1, NW, RB), lambda i: (i, 0, 0)),
        ],
        out_specs=[
            pl.BlockSpec((RB, DH), lambda i: (i, 0)),
            pl.BlockSpec((RB, DH), lambda i: (i, 0)),
        ],
        out_shape=[
            jax.ShapeDtypeStruct((N_NODES, DH), jnp.float32),
            jax.ShapeDtypeStruct((N_NODES, DH), jnp.float32),
        ],
    )(v, W, degp)


EPT = N_EDGES // NS
CK = 80
NCHUNK = EPT // CK
STRIPE = 640
N_PAD = STRIPE * NS
FR = CK


def _agg_body(xl_hbm, xr_hbm, e3_hbm, outl_hbm, outr_hbm,
              idx0_v, idx1_v, idx2_v, rows0_v, rows1_v, rows2_v, acc_sh,
              gsem0, gsem1, gsem2, ssem0, ssem1, ssem2,
              isem0, isem1, isem2):
    c = lax.axis_index("c")
    s = lax.axis_index("s")
    zeros16 = jnp.zeros((L,), jnp.float32)
    idx = (idx0_v, idx1_v, idx2_v)
    rows = (rows0_v, rows1_v, rows2_v)
    gsem = (gsem0, gsem1, gsem2)
    ssem = (ssem0, ssem1, ssem2)
    isem = (isem0, isem1, isem2)

    def zb_step(i, _):
        for j in range(DH // L):
            rows0_v[i, pl.ds(j * L, L)] = zeros16
        return 0

    lax.fori_loop(0, FR, zb_step, 0)
    sbase = pl.multiple_of(s * STRIPE, 8)

    def zs_step(i, _):
        pltpu.sync_copy(rows0_v, acc_sh.at[pl.ds(sbase + i * FR, FR)])
        return 0

    lax.fori_loop(0, STRIPE // FR, zs_step, 0)
    plsc.subcore_barrier()

    def run(x_hbm):
        def start_idx(j, m):
            pltpu.async_copy(e3_hbm.at[s * NCHUNK + j], idx[m], isem[m])

        def wait_idx(j, m):
            pltpu.make_async_copy(e3_hbm.at[s * NCHUNK + j], idx[m],
                                  isem[m]).wait()

        def start_gather(j, m):
            pltpu.async_copy(x_hbm.at[idx[m].at[0]], rows[m], gsem[m])

        def wait_gather(j, m):
            pltpu.make_async_copy(x_hbm.at[idx[m].at[0]], rows[m],
                                  gsem[m]).wait()

        def start_scatter(j, m):
            pltpu.async_copy(rows[m], acc_sh.at[idx[m].at[1]], ssem[m],
                             add=True)

        def wait_scatter(j, m):
            pltpu.make_async_copy(rows[m], acc_sh.at[idx[m].at[1]],
                                  ssem[m]).wait()

        def body(j, m, first, idx_ahead, gather_ahead):
            mn, mf = (m + 1) % 3, (m + 2) % 3
            if not first:
                wait_scatter(j - 1, mf)
            if idx_ahead:
                start_idx(j + 2, mf)
            if gather_ahead:
                wait_idx(j + 1, mn)
                start_gather(j + 1, mn)
            wait_gather(j, m)
            start_scatter(j, m)

        start_idx(0, 0)
        start_idx(1, 1)
        wait_idx(0, 0)
        start_gather(0, 0)

        def tri_step(g, _):
            for p in range(3):
                j3 = 3 * g + p

                @pl.when(j3 > 0)
                def _():
                    wait_scatter(j3 - 1, (p + 2) % 3)

                start_idx(j3 + 2, (p + 2) % 3)
                wait_idx(j3 + 1, (p + 1) % 3)
                start_gather(j3 + 1, (p + 1) % 3)
                wait_gather(j3, p)
                start_scatter(j3, p)
            return 0

        nloop = (NCHUNK - 2) // 3
        lax.fori_loop(0, nloop, tri_step, 0)
        for j in range(3 * nloop, NCHUNK):
            m = j % 3
            body(j, m, first=False, idx_ahead=False,
                 gather_ahead=(j + 1 < NCHUNK))
        wait_scatter(NCHUNK - 1, (NCHUNK - 1) % 3)

    def flush(out_hbm):
        def f_step(i, _):
            off = pl.multiple_of(sbase + i * FR, 8)
            pltpu.sync_copy(acc_sh.at[pl.ds(off, FR)], rows0_v)
            pltpu.sync_copy(rows0_v, out_hbm.at[pl.ds(off, FR)])
            return 0

        lax.fori_loop(0, STRIPE // FR, f_step, 0)

    @pl.when(c == 0)
    def _():
        run(xl_hbm)

    @pl.when(c == 1)
    def _():
        run(xr_hbm)

    plsc.subcore_barrier()

    @pl.when(c == 0)
    def _():
        flush(outl_hbm)

    @pl.when(c == 1)
    def _():
        flush(outr_hbm)


def _edge_agg(xl, xr, e3):
    mesh = plsc.VectorSubcoreMesh(core_axis_name="c", subcore_axis_name="s",
                                  num_cores=NC, num_subcores=NS)
    return pl.kernel(
        _agg_body,
        out_type=(jax.ShapeDtypeStruct((N_PAD, DH), jnp.float32),
                  jax.ShapeDtypeStruct((N_PAD, DH), jnp.float32)),
        mesh=mesh,
        scratch_types=(
            [pltpu.VMEM((2, CK), jnp.int32)] * 3
            + [pltpu.VMEM((CK, DH), jnp.float32)] * 3
            + [pltpu.VMEM_SHARED((N_PAD, DH), jnp.float32)]
            + [pltpu.SemaphoreType.DMA] * 9
        ),
        compiler_params=pltpu.CompilerParams(needs_layout_passes=False),
    )(xl, xr, e3)


def _pool_body(aggl_ref, aggr_ref, xl_ref, xr_ref, degp_ref, b_ref, batch_ref,
               out_ref, sum_acc, cnt_acc):
    i = pl.program_id(0)
    deg = jnp.sum(degp_ref[0], axis=0) + 1.0
    dinv = lax.rsqrt(deg)
    agg = jnp.concatenate([aggl_ref[...], aggr_ref[...]], axis=1)
    xs = jnp.concatenate([xl_ref[...], xr_ref[...]], axis=1)
    h = jnp.maximum(dinv[:, None] * (agg + xs) + b_ref[...], 0.0)
    bvec = batch_ref[...].reshape(1, RB)
    onehot = (lax.broadcasted_iota(jnp.int32, (N_GRAPHS, RB), 0)
              == bvec).astype(jnp.float32)

    @pl.when(i == 0)
    def _():
        sum_acc[...] = jnp.zeros_like(sum_acc)
        cnt_acc[...] = jnp.zeros_like(cnt_acc)

    sum_acc[...] += jnp.dot(onehot, h, preferred_element_type=jnp.float32)
    cnt_acc[...] += jnp.dot(onehot, jnp.ones((RB, D_OUT), jnp.float32),
                            preferred_element_type=jnp.float32)

    @pl.when(i == NRB - 1)
    def _():
        out_ref[...] = sum_acc[...] / jnp.maximum(cnt_acc[...], 1.0)


def _pool(aggl, aggr, xl, xr, degp, b2, batch3):
    return pl.pallas_call(
        _pool_body,
        grid=(NRB,),
        in_specs=[
            pl.BlockSpec((RB, DH), lambda i: (i, 0)),
            pl.BlockSpec((RB, DH), lambda i: (i, 0)),
            pl.BlockSpec((RB, DH), lambda i: (i, 0)),
            pl.BlockSpec((RB, DH), lambda i: (i, 0)),
            pl.BlockSpec((1, NW, RB), lambda i: (i, 0, 0)),
            pl.BlockSpec((1, D_OUT), lambda i: (0, 0)),
            pl.BlockSpec((1, 1, RB), lambda i: (i, 0, 0)),
        ],
        out_specs=pl.BlockSpec((N_GRAPHS, D_OUT), lambda i: (0, 0)),
        out_shape=jax.ShapeDtypeStruct((N_GRAPHS, D_OUT), jnp.float32),
        scratch_shapes=[
            pltpu.VMEM((N_GRAPHS, D_OUT), jnp.float32),
            pltpu.VMEM((N_GRAPHS, D_OUT), jnp.float32),
        ],
    )(aggl, aggr, xl, xr, degp, b2, batch3)


def kernel(v, e, batch, W, b):
    e = e.astype(jnp.int32)
    row, col = e[0], e[1]
    pad = NW * EPW + PAD_E - N_EDGES
    col_padded = jnp.concatenate([col, jnp.zeros((pad,), jnp.int32)])
    degp = _deg_partials(col_padded)
    degp_t = degp.reshape(NW, NRB, RB).transpose(1, 0, 2)
    xl, xr = _scaled_x(v, W, degp_t)
    e3 = (e.reshape(2, NS, NCHUNK, CK).transpose(1, 2, 0, 3)
          .reshape(NS * NCHUNK, 2, CK))
    aggl, aggr = _edge_agg(xl, xr, e3)
    return _pool(aggl, aggr, xl, xr, degp_t, b.reshape(1, D_OUT),
                 batch.astype(jnp.int32).reshape(NRB, 1, RB))


# speedup vs baseline: 24.9715x; 1.0057x over previous
import functools

import jax
import jax.numpy as jnp
from jax import lax
from jax.experimental import pallas as pl
from jax.experimental.pallas import tpu as pltpu
from jax.experimental.pallas import tpu_sc as plsc

N_NODES = 10000
N_EDGES = 160000
D_IN = 256
D_OUT = 256
N_GRAPHS = 128
DH = 128
NC = 2
NS = 16
NW = NC * NS
L = 16

EPW = N_EDGES // NW
FULL_VREGS = EPW // L
TAIL = EPW - FULL_VREGS * L
PAD_E = FULL_VREGS * L + L


def _deg_body(col_hbm, degp_hbm, idx_v, deg_v):
    c = lax.axis_index("c")
    s = lax.axis_index("s")
    w = s * NC + c
    zeros16 = jnp.zeros((L,), jnp.float32)
    ones16 = jnp.ones((L,), jnp.float32)

    def zero_step(i, _):
        deg_v[pl.ds(i * L, L)] = zeros16
        return 0

    lax.fori_loop(0, N_NODES // L, zero_step, 0)
    pltpu.sync_copy(col_hbm.at[pl.ds(w * EPW, PAD_E)], idx_v)

    def hist_step(j, _):
        idx = idx_v[pl.ds(j * L, L)]
        plsc.addupdate_scatter(deg_v, [idx], ones16)
        return 0

    lax.fori_loop(0, FULL_VREGS, hist_step, 0)
    tail_idx = idx_v[pl.ds(FULL_VREGS * L, L)]
    tail_mask = lax.iota(jnp.int32, L) < TAIL
    plsc.addupdate_scatter(deg_v, [tail_idx], ones16, mask=tail_mask)

    def out_step(i, _):
        pltpu.sync_copy(deg_v.at[pl.ds(i * RB, RB)],
                        degp_hbm.at[pl.ds(i * (NW * RB) + w * RB, RB)])
        return 0

    lax.fori_loop(0, NRB, out_step, 0)


def _deg_partials(col_padded):
    mesh = plsc.VectorSubcoreMesh(core_axis_name="c", subcore_axis_name="s",
                                  num_cores=NC, num_subcores=NS)
    return pl.kernel(
        _deg_body,
        out_type=jax.ShapeDtypeStruct((NW * N_NODES,), jnp.float32),
        mesh=mesh,
        scratch_types=[
            pltpu.VMEM((PAD_E,), jnp.int32),
            pltpu.VMEM((N_NODES,), jnp.float32),
        ],
        compiler_params=pltpu.CompilerParams(needs_layout_passes=False),
    )(col_padded)


RB = 1000
NRB = N_NODES // RB


def _scale_mm_body(v_ref, w_ref, degp_ref, xl_ref, xr_ref):
    x = jnp.dot(v_ref[...], w_ref[...], preferred_element_type=jnp.float32)
    deg = jnp.sum(degp_ref[0], axis=0) + 1.0
    dinv = lax.rsqrt(deg)
    xs = x * dinv[:, None]
    xl_ref[...] = xs[:, :DH]
    xr_ref[...] = xs[:, DH:]


def _scaled_x(v, W, degp):
    return pl.pallas_call(
        _scale_mm_body,
        grid=(NRB,),
        in_specs=[
            pl.BlockSpec((RB, D_IN), lambda i: (i, 0)),
            pl.BlockSpec((D_IN, D_OUT), lambda i: (0, 0)),
            pl.BlockSpec((1, NW, RB), lambda i: (i, 0, 0)),
        ],
        out_specs=[
            pl.BlockSpec((RB, DH), lambda i: (i, 0)),
            pl.BlockSpec((RB, DH), lambda i: (i, 0)),
        ],
        out_shape=[
            jax.ShapeDtypeStruct((N_NODES, DH), jnp.float32),
            jax.ShapeDtypeStruct((N_NODES, DH), jnp.float32),
        ],
    )(v, W, degp)


EPT = N_EDGES // NS
CK = 80
NCHUNK = EPT // CK
STRIPE = 640
N_PAD = STRIPE * NS
FR = CK


def _agg_body(xl_hbm, xr_hbm, e3_hbm, outl_hbm, outr_hbm,
              idx0_v, idx1_v, idx2_v, rows0_v, rows1_v, rows2_v, acc_sh,
              gsem0, gsem1, gsem2, ssem0, ssem1, ssem2,
              isem0, isem1, isem2):
    c = lax.axis_index("c")
    s = lax.axis_index("s")
    zeros16 = jnp.zeros((L,), jnp.float32)
    idx = (idx0_v, idx1_v, idx2_v)
    rows = (rows0_v, rows1_v, rows2_v)
    gsem = (gsem0, gsem1, gsem2)
    ssem = (ssem0, ssem1, ssem2)
    isem = (isem0, isem1, isem2)

    def zb_step(i, _):
        for j in range(DH // L):
            rows0_v[i, pl.ds(j * L, L)] = zeros16
        return 0

    lax.fori_loop(0, FR, zb_step, 0)
    sbase = pl.multiple_of(s * STRIPE, 8)

    def zs_step(i, _):
        pltpu.sync_copy(rows0_v, acc_sh.at[pl.ds(sbase + i * FR, FR)])
        return 0

    lax.fori_loop(0, STRIPE // FR, zs_step, 0)
    plsc.subcore_barrier()

    def run(x_hbm):
        def start_idx(j, m):
            pltpu.async_copy(e3_hbm.at[s * NCHUNK + j], idx[m], isem[m])

        def wait_idx(j, m):
            pltpu.make_async_copy(e3_hbm.at[s * NCHUNK + j], idx[m],
                                  isem[m]).wait()

        def start_gather(j, m):
            pltpu.async_copy(x_hbm.at[idx[m].at[0]], rows[m], gsem[m])

        def wait_gather(j, m):
            pltpu.make_async_copy(x_hbm.at[idx[m].at[0]], rows[m],
                                  gsem[m]).wait()

        def start_scatter(j, m):
            pltpu.async_copy(rows[m], acc_sh.at[idx[m].at[1]], ssem[m],
                             add=True)

        def wait_scatter(j, m):
            pltpu.make_async_copy(rows[m], acc_sh.at[idx[m].at[1]],
                                  ssem[m]).wait()

        def body(j, m, first, idx_ahead, gather_ahead):
            mn, mf = (m + 1) % 3, (m + 2) % 3
            if not first:
                wait_scatter(j - 1, mf)
            if idx_ahead:
                start_idx(j + 2, mf)
            if gather_ahead:
                wait_idx(j + 1, mn)
                start_gather(j + 1, mn)
            wait_gather(j, m)
            start_scatter(j, m)

        start_idx(0, 0)
        start_idx(1, 1)
        wait_idx(0, 0)
        start_gather(0, 0)

        def tri_step(g, _):
            for p in range(3):
                j3 = 3 * g + p

                @pl.when(j3 > 0)
                def _():
                    wait_scatter(j3 - 1, (p + 2) % 3)

                start_idx(j3 + 2, (p + 2) % 3)
                wait_idx(j3 + 1, (p + 1) % 3)
                start_gather(j3 + 1, (p + 1) % 3)
                wait_gather(j3, p)
                start_scatter(j3, p)
            return 0

        nloop = (NCHUNK - 2) // 3
        lax.fori_loop(0, nloop, tri_step, 0)
        for j in range(3 * nloop, NCHUNK):
            m = j % 3
            body(j, m, first=False, idx_ahead=False,
                 gather_ahead=(j + 1 < NCHUNK))
        wait_scatter(NCHUNK - 1, (NCHUNK - 1) % 3)

    def flush(out_hbm):
        def f_step(i, _):
            off = pl.multiple_of(sbase + i * FR, 8)
            pltpu.sync_copy(acc_sh.at[pl.ds(off, FR)], rows0_v)
            pltpu.sync_copy(rows0_v, out_hbm.at[pl.ds(off, FR)])
            return 0

        lax.fori_loop(0, STRIPE // FR, f_step, 0)

    @pl.when(c == 0)
    def _():
        run(xl_hbm)

    @pl.when(c == 1)
    def _():
        run(xr_hbm)

    plsc.subcore_barrier()

    @pl.when(c == 0)
    def _():
        flush(outl_hbm)

    @pl.when(c == 1)
    def _():
        flush(outr_hbm)


def _edge_agg(xl, xr, e3):
    mesh = plsc.VectorSubcoreMesh(core_axis_name="c", subcore_axis_name="s",
                                  num_cores=NC, num_subcores=NS)
    return pl.kernel(
        _agg_body,
        out_type=(jax.ShapeDtypeStruct((N_PAD, DH), jnp.float32),
                  jax.ShapeDtypeStruct((N_PAD, DH), jnp.float32)),
        mesh=mesh,
        scratch_types=(
            [pltpu.VMEM((2, CK), jnp.int32)] * 3
            + [pltpu.VMEM((CK, DH), jnp.float32)] * 3
            + [pltpu.VMEM_SHARED((N_PAD, DH), jnp.float32)]
            + [pltpu.SemaphoreType.DMA] * 9
        ),
        compiler_params=pltpu.CompilerParams(needs_layout_passes=False),
    )(xl, xr, e3)


def _pool_body(aggl_ref, aggr_ref, xl_ref, xr_ref, degp_ref, b_ref, batch_ref,
               out_ref, sum_acc, cnt_acc):
    i = pl.program_id(0)
    deg = jnp.sum(degp_ref[0], axis=0) + 1.0
    dinv = lax.rsqrt(deg)
    agg = jnp.concatenate([aggl_ref[...], aggr_ref[...]], axis=1)
    xs = jnp.concatenate([xl_ref[...], xr_ref[...]], axis=1)
    h = jnp.maximum(dinv[:, None] * (agg + xs) + b_ref[...], 0.0)
    bvec = batch_ref[...].reshape(1, RB)
    onehot = (lax.broadcasted_iota(jnp.int32, (N_GRAPHS, RB), 0)
              == bvec).astype(jnp.float32)

    @pl.when(i == 0)
    def _():
        sum_acc[...] = jnp.zeros_like(sum_acc)
        cnt_acc[...] = jnp.zeros_like(cnt_acc)

    sum_acc[...] += jnp.dot(onehot, h, preferred_element_type=jnp.float32)
    cnt_acc[...] += jnp.dot(onehot, jnp.ones((RB, D_OUT), jnp.float32),
                            preferred_element_type=jnp.float32)

    @pl.when(i == NRB - 1)
    def _():
        out_ref[...] = sum_acc[...] / jnp.maximum(cnt_acc[...], 1.0)


def _pool(aggl, aggr, xl, xr, degp, b2, batch3):
    return pl.pallas_call(
        _pool_body,
        grid=(NRB,),
        in_specs=[
            pl.BlockSpec((RB, DH), lambda i: (i, 0)),
            pl.BlockSpec((RB, DH), lambda i: (i, 0)),
            pl.BlockSpec((RB, DH), lambda i: (i, 0)),
            pl.BlockSpec((RB, DH), lambda i: (i, 0)),
            pl.BlockSpec((1, NW, RB), lambda i: (i, 0, 0)),
            pl.BlockSpec((1, D_OUT), lambda i: (0, 0)),
            pl.BlockSpec((1, 1, RB), lambda i: (i, 0, 0)),
        ],
        out_specs=pl.BlockSpec((N_GRAPHS, D_OUT), lambda i: (0, 0)),
        out_shape=jax.ShapeDtypeStruct((N_GRAPHS, D_OUT), jnp.float32),
        scratch_shapes=[
            pltpu.VMEM((N_GRAPHS, D_OUT), jnp.float32),
            pltpu.VMEM((N_GRAPHS, D_OUT), jnp.float32),
        ],
    )(aggl, aggr, xl, xr, degp, b2, batch3)


def kernel(v, e, batch, W, b):
    e = e.astype(jnp.int32)
    row, col = e[0], e[1]
    pad = NW * EPW + PAD_E - N_EDGES
    col_padded = jnp.concatenate([col, jnp.zeros((pad,), jnp.int32)])
    degp = _deg_partials(col_padded)
    degp_t = degp.reshape(NRB, NW, RB)
    xl, xr = _scaled_x(v, W, degp_t)
    e3 = (e.reshape(2, NS, NCHUNK, CK).transpose(1, 2, 0, 3)
          .reshape(NS * NCHUNK, 2, CK))
    aggl, aggr = _edge_agg(xl, xr, e3)
    return _pool(aggl, aggr, xl, xr, degp_t, b.reshape(1, D_OUT),
                 batch.astype(jnp.int32).reshape(NRB, 1, RB))


# speedup vs baseline: 25.8625x; 1.0357x over previous
import functools

import jax
import jax.numpy as jnp
from jax import lax
from jax.experimental import pallas as pl
from jax.experimental.pallas import tpu as pltpu
from jax.experimental.pallas import tpu_sc as plsc

N_NODES = 10000
N_EDGES = 160000
D_IN = 256
D_OUT = 256
N_GRAPHS = 128
DH = 128
NC = 2
NS = 16
NW = NC * NS
L = 16

EPW = N_EDGES // NW
FULL_VREGS = EPW // L
TAIL = EPW - FULL_VREGS * L
PAD_E = FULL_VREGS * L + L


def _deg_body(col_hbm, degp_hbm, idx_v, deg_v):
    c = lax.axis_index("c")
    s = lax.axis_index("s")
    w = s * NC + c
    zeros16 = jnp.zeros((L,), jnp.float32)
    ones16 = jnp.ones((L,), jnp.float32)

    def zero_step(i, _):
        deg_v[pl.ds(i * L, L)] = zeros16
        return 0

    lax.fori_loop(0, N_NODES // L, zero_step, 0)
    pltpu.sync_copy(col_hbm.at[pl.ds(w * EPW, PAD_E)], idx_v)

    def hist_step(j, _):
        idx = idx_v[pl.ds(j * L, L)]
        plsc.addupdate_scatter(deg_v, [idx], ones16)
        return 0

    lax.fori_loop(0, FULL_VREGS, hist_step, 0)
    tail_idx = idx_v[pl.ds(FULL_VREGS * L, L)]
    tail_mask = lax.iota(jnp.int32, L) < TAIL
    plsc.addupdate_scatter(deg_v, [tail_idx], ones16, mask=tail_mask)

    def out_step(i, _):
        pltpu.sync_copy(deg_v.at[pl.ds(i * RB, RB)],
                        degp_hbm.at[pl.ds(i * (NW * RB) + w * RB, RB)])
        return 0

    lax.fori_loop(0, NRB, out_step, 0)


def _deg_partials(col_padded):
    mesh = plsc.VectorSubcoreMesh(core_axis_name="c", subcore_axis_name="s",
                                  num_cores=NC, num_subcores=NS)
    return pl.kernel(
        _deg_body,
        out_type=jax.ShapeDtypeStruct((NW * N_NODES,), jnp.float32),
        mesh=mesh,
        scratch_types=[
            pltpu.VMEM((PAD_E,), jnp.int32),
            pltpu.VMEM((N_NODES,), jnp.float32),
        ],
        compiler_params=pltpu.CompilerParams(needs_layout_passes=False),
    )(col_padded)


RB = 1000
NRB = N_NODES // RB


def _scale_mm_body(v_ref, w_ref, degp_ref, xl_ref, xr_ref):
    x = jnp.dot(v_ref[...], w_ref[...], preferred_element_type=jnp.float32)
    deg = jnp.sum(degp_ref[0], axis=0) + 1.0
    dinv = lax.rsqrt(deg)
    xs = x * dinv[:, None]
    xl_ref[...] = xs[:, :DH]
    xr_ref[...] = xs[:, DH:]


def _scaled_x(v, W, degp):
    return pl.pallas_call(
        _scale_mm_body,
        grid=(NRB,),
        in_specs=[
            pl.BlockSpec((RB, D_IN), lambda i: (i, 0)),
            pl.BlockSpec((D_IN, D_OUT), lambda i: (0, 0)),
            pl.BlockSpec((1, NW, RB), lambda i: (i, 0, 0)),
        ],
        out_specs=[
            pl.BlockSpec((RB, DH), lambda i: (i, 0)),
            pl.BlockSpec((RB, DH), lambda i: (i, 0)),
        ],
        out_shape=[
            jax.ShapeDtypeStruct((N_NODES, DH), jnp.float32),
            jax.ShapeDtypeStruct((N_NODES, DH), jnp.float32),
        ],
    )(v, W, degp)


EPT = N_EDGES // NS
CK = 80
NCHUNK = EPT // CK
STRIPE = 640
N_PAD = STRIPE * NS
FR = CK


def _agg_body(xl_hbm, xr_hbm, e3_hbm, outl_hbm, outr_hbm,
              idx0_v, idx1_v, idx2_v, rows0_v, rows1_v, rows2_v, acc_sh,
              gsem0, gsem1, gsem2, ssem0, ssem1, ssem2,
              isem0, isem1, isem2):
    c = lax.axis_index("c")
    s = lax.axis_index("s")
    zeros16 = jnp.zeros((L,), jnp.float32)
    idx = (idx0_v, idx1_v, idx2_v)
    rows = (rows0_v, rows1_v, rows2_v)
    gsem = (gsem0, gsem1, gsem2)
    ssem = (ssem0, ssem1, ssem2)
    isem = (isem0, isem1, isem2)

    def zb_step(i, _):
        for j in range(DH // L):
            rows0_v[i, pl.ds(j * L, L)] = zeros16
        return 0

    lax.fori_loop(0, FR, zb_step, 0)
    sbase = pl.multiple_of(s * STRIPE, 8)

    def zs_step(i, _):
        pltpu.sync_copy(rows0_v, acc_sh.at[pl.ds(sbase + i * FR, FR)])
        return 0

    lax.fori_loop(0, STRIPE // FR, zs_step, 0)
    plsc.subcore_barrier()

    def run(x_hbm):
        def start_idx(j, m):
            pltpu.async_copy(e3_hbm.at[s * NCHUNK + j], idx[m], isem[m])

        def wait_idx(j, m):
            pltpu.make_async_copy(e3_hbm.at[s * NCHUNK + j], idx[m],
                                  isem[m]).wait()

        def start_gather(j, m):
            pltpu.async_copy(x_hbm.at[idx[m].at[0]], rows[m], gsem[m])

        def wait_gather(j, m):
            pltpu.make_async_copy(x_hbm.at[idx[m].at[0]], rows[m],
                                  gsem[m]).wait()

        def start_scatter(j, m):
            pltpu.async_copy(rows[m], acc_sh.at[idx[m].at[1]], ssem[m],
                             add=True)

        def wait_scatter(j, m):
            pltpu.make_async_copy(rows[m], acc_sh.at[idx[m].at[1]],
                                  ssem[m]).wait()

        def body(j, m, first, idx_ahead, gather_ahead):
            mn, mf = (m + 1) % 3, (m + 2) % 3
            if gather_ahead:
                wait_idx(j + 1, mn)
                start_gather(j + 1, mn)
            if not first:
                wait_scatter(j - 1, mf)
            if idx_ahead:
                start_idx(j + 2, mf)
            wait_gather(j, m)
            start_scatter(j, m)

        start_idx(0, 0)
        start_idx(1, 1)
        wait_idx(0, 0)
        start_gather(0, 0)

        def tri_step(g, _):
            for p in range(3):
                j3 = 3 * g + p
                wait_idx(j3 + 1, (p + 1) % 3)
                start_gather(j3 + 1, (p + 1) % 3)

                @pl.when(j3 > 0)
                def _():
                    wait_scatter(j3 - 1, (p + 2) % 3)

                start_idx(j3 + 2, (p + 2) % 3)
                wait_gather(j3, p)
                start_scatter(j3, p)
            return 0

        nloop = (NCHUNK - 2) // 3
        lax.fori_loop(0, nloop, tri_step, 0)
        for j in range(3 * nloop, NCHUNK):
            m = j % 3
            body(j, m, first=False, idx_ahead=(j + 2 < NCHUNK),
                 gather_ahead=(j + 1 < NCHUNK))
        wait_scatter(NCHUNK - 1, (NCHUNK - 1) % 3)

    def flush(out_hbm):
        def f_step(i, _):
            off = pl.multiple_of(sbase + i * FR, 8)
            pltpu.sync_copy(acc_sh.at[pl.ds(off, FR)], rows0_v)
            pltpu.sync_copy(rows0_v, out_hbm.at[pl.ds(off, FR)])
            return 0

        lax.fori_loop(0, STRIPE // FR, f_step, 0)

    @pl.when(c == 0)
    def _():
        run(xl_hbm)

    @pl.when(c == 1)
    def _():
        run(xr_hbm)

    plsc.subcore_barrier()

    @pl.when(c == 0)
    def _():
        flush(outl_hbm)

    @pl.when(c == 1)
    def _():
        flush(outr_hbm)


def _edge_agg(xl, xr, e3):
    mesh = plsc.VectorSubcoreMesh(core_axis_name="c", subcore_axis_name="s",
                                  num_cores=NC, num_subcores=NS)
    return pl.kernel(
        _agg_body,
        out_type=(jax.ShapeDtypeStruct((N_PAD, DH), jnp.float32),
                  jax.ShapeDtypeStruct((N_PAD, DH), jnp.float32)),
        mesh=mesh,
        scratch_types=(
            [pltpu.VMEM((2, CK), jnp.int32)] * 3
            + [pltpu.VMEM((CK, DH), jnp.float32)] * 3
            + [pltpu.VMEM_SHARED((N_PAD, DH), jnp.float32)]
            + [pltpu.SemaphoreType.DMA] * 9
        ),
        compiler_params=pltpu.CompilerParams(needs_layout_passes=False),
    )(xl, xr, e3)


def _pool_body(aggl_ref, aggr_ref, xl_ref, xr_ref, degp_ref, b_ref, batch_ref,
               out_ref, sum_acc, cnt_acc):
    i = pl.program_id(0)
    deg = jnp.sum(degp_ref[0], axis=0) + 1.0
    dinv = lax.rsqrt(deg)
    agg = jnp.concatenate([aggl_ref[...], aggr_ref[...]], axis=1)
    xs = jnp.concatenate([xl_ref[...], xr_ref[...]], axis=1)
    h = jnp.maximum(dinv[:, None] * (agg + xs) + b_ref[...], 0.0)
    bvec = batch_ref[...].reshape(1, RB)
    onehot = (lax.broadcasted_iota(jnp.int32, (N_GRAPHS, RB), 0)
              == bvec).astype(jnp.float32)

    @pl.when(i == 0)
    def _():
        sum_acc[...] = jnp.zeros_like(sum_acc)
        cnt_acc[...] = jnp.zeros_like(cnt_acc)

    sum_acc[...] += jnp.dot(onehot, h, preferred_element_type=jnp.float32)
    cnt_acc[...] += jnp.dot(onehot, jnp.ones((RB, D_OUT), jnp.float32),
                            preferred_element_type=jnp.float32)

    @pl.when(i == NRB - 1)
    def _():
        out_ref[...] = sum_acc[...] / jnp.maximum(cnt_acc[...], 1.0)


def _pool(aggl, aggr, xl, xr, degp, b2, batch3):
    return pl.pallas_call(
        _pool_body,
        grid=(NRB,),
        in_specs=[
            pl.BlockSpec((RB, DH), lambda i: (i, 0)),
            pl.BlockSpec((RB, DH), lambda i: (i, 0)),
            pl.BlockSpec((RB, DH), lambda i: (i, 0)),
            pl.BlockSpec((RB, DH), lambda i: (i, 0)),
            pl.BlockSpec((1, NW, RB), lambda i: (i, 0, 0)),
            pl.BlockSpec((1, D_OUT), lambda i: (0, 0)),
            pl.BlockSpec((1, 1, RB), lambda i: (i, 0, 0)),
        ],
        out_specs=pl.BlockSpec((N_GRAPHS, D_OUT), lambda i: (0, 0)),
        out_shape=jax.ShapeDtypeStruct((N_GRAPHS, D_OUT), jnp.float32),
        scratch_shapes=[
            pltpu.VMEM((N_GRAPHS, D_OUT), jnp.float32),
            pltpu.VMEM((N_GRAPHS, D_OUT), jnp.float32),
        ],
    )(aggl, aggr, xl, xr, degp, b2, batch3)


def kernel(v, e, batch, W, b):
    e = e.astype(jnp.int32)
    row, col = e[0], e[1]
    pad = NW * EPW + PAD_E - N_EDGES
    col_padded = jnp.concatenate([col, jnp.zeros((pad,), jnp.int32)])
    degp = _deg_partials(col_padded)
    degp_t = degp.reshape(NRB, NW, RB)
    xl, xr = _scaled_x(v, W, degp_t)
    e3 = (e.reshape(2, NS, NCHUNK, CK).transpose(1, 2, 0, 3)
          .reshape(NS * NCHUNK, 2, CK))
    aggl, aggr = _edge_agg(xl, xr, e3)
    return _pool(aggl, aggr, xl, xr, degp_t, b.reshape(1, D_OUT),
                 batch.astype(jnp.int32).reshape(NRB, 1, RB))


# speedup vs baseline: 27.5687x; 1.0660x over previous
import functools

import jax
import jax.numpy as jnp
from jax import lax
from jax.experimental import pallas as pl
from jax.experimental.pallas import tpu as pltpu
from jax.experimental.pallas import tpu_sc as plsc

N_NODES = 10000
N_EDGES = 160000
D_IN = 256
D_OUT = 256
N_GRAPHS = 128
DH = 128
NC = 2
NS = 16
NW = NC * NS
L = 16

EPW = N_EDGES // NW
FULL_VREGS = EPW // L
TAIL = EPW - FULL_VREGS * L
PAD_E = FULL_VREGS * L + L


def _deg_body(col_hbm, degp_hbm, idx_v, deg_v):
    c = lax.axis_index("c")
    s = lax.axis_index("s")
    w = s * NC + c
    zeros16 = jnp.zeros((L,), jnp.float32)
    ones16 = jnp.ones((L,), jnp.float32)

    def zero_step(i, _):
        deg_v[pl.ds(i * L, L)] = zeros16
        return 0

    lax.fori_loop(0, N_NODES // L, zero_step, 0)
    pltpu.sync_copy(col_hbm.at[pl.ds(w * EPW, PAD_E)], idx_v)

    def hist_step(j, _):
        idx = idx_v[pl.ds(j * L, L)]
        plsc.addupdate_scatter(deg_v, [idx], ones16)
        return 0

    lax.fori_loop(0, FULL_VREGS, hist_step, 0)
    tail_idx = idx_v[pl.ds(FULL_VREGS * L, L)]
    tail_mask = lax.iota(jnp.int32, L) < TAIL
    plsc.addupdate_scatter(deg_v, [tail_idx], ones16, mask=tail_mask)

    def out_step(i, _):
        pltpu.sync_copy(deg_v.at[pl.ds(i * RB, RB)],
                        degp_hbm.at[pl.ds(i * (NW * RB) + w * RB, RB)])
        return 0

    lax.fori_loop(0, NRB, out_step, 0)


def _deg_partials(col_padded):
    mesh = plsc.VectorSubcoreMesh(core_axis_name="c", subcore_axis_name="s",
                                  num_cores=NC, num_subcores=NS)
    return pl.kernel(
        _deg_body,
        out_type=jax.ShapeDtypeStruct((NW * N_NODES,), jnp.float32),
        mesh=mesh,
        scratch_types=[
            pltpu.VMEM((PAD_E,), jnp.int32),
            pltpu.VMEM((N_NODES,), jnp.float32),
        ],
        compiler_params=pltpu.CompilerParams(needs_layout_passes=False),
    )(col_padded)


RB = 1000
NRB = N_NODES // RB


def _scale_mm_body(v_ref, w_ref, degp_ref, xl_ref, xr_ref):
    x = jnp.dot(v_ref[...], w_ref[...], preferred_element_type=jnp.float32)
    deg = jnp.sum(degp_ref[0], axis=0) + 1.0
    dinv = lax.rsqrt(deg)
    xs = x * dinv[:, None]
    xl_ref[...] = xs[:, :DH]
    xr_ref[...] = xs[:, DH:]


def _scaled_x(v, W, degp):
    return pl.pallas_call(
        _scale_mm_body,
        grid=(NRB,),
        in_specs=[
            pl.BlockSpec((RB, D_IN), lambda i: (i, 0)),
            pl.BlockSpec((D_IN, D_OUT), lambda i: (0, 0)),
            pl.BlockSpec((1, NW, RB), lambda i: (i, 0, 0)),
        ],
        out_specs=[
            pl.BlockSpec((RB, DH), lambda i: (i, 0)),
            pl.BlockSpec((RB, DH), lambda i: (i, 0)),
        ],
        out_shape=[
            jax.ShapeDtypeStruct((N_NODES, DH), jnp.float32),
            jax.ShapeDtypeStruct((N_NODES, DH), jnp.float32),
        ],
    )(v, W, degp)


EPT = N_EDGES // NS
CK = 80
NCHUNK = EPT // CK
STRIPE = 640
N_PAD = STRIPE * NS
FR = CK


def _agg_body(xl_hbm, xr_hbm, e3_hbm, outl_hbm, outr_hbm,
              idx0_v, idx1_v, idx2_v, idx3_v, rows0_v, rows1_v, rows2_v,
              acc_sh, gsem0, gsem1, gsem2, ssem0, ssem1, ssem2,
              isem0, isem1, isem2, isem3):
    c = lax.axis_index("c")
    s = lax.axis_index("s")
    zeros16 = jnp.zeros((L,), jnp.float32)
    idx = (idx0_v, idx1_v, idx2_v, idx3_v)
    rows = (rows0_v, rows1_v, rows2_v)
    gsem = (gsem0, gsem1, gsem2)
    ssem = (ssem0, ssem1, ssem2)
    isem = (isem0, isem1, isem2, isem3)

    def zb_step(i, _):
        for j in range(DH // L):
            rows0_v[i, pl.ds(j * L, L)] = zeros16
        return 0

    lax.fori_loop(0, FR, zb_step, 0)
    sbase = pl.multiple_of(s * STRIPE, 8)

    def zs_step(i, _):
        pltpu.sync_copy(rows0_v, acc_sh.at[pl.ds(sbase + i * FR, FR)])
        return 0

    lax.fori_loop(0, STRIPE // FR, zs_step, 0)
    plsc.subcore_barrier()

    def run(x_hbm):
        def start_idx(j, m):
            pltpu.async_copy(e3_hbm.at[s * NCHUNK + j], idx[m], isem[m])

        def wait_idx(j, m):
            pltpu.make_async_copy(e3_hbm.at[s * NCHUNK + j], idx[m],
                                  isem[m]).wait()

        def start_gather(j, ri, ii):
            pltpu.async_copy(x_hbm.at[idx[ii].at[0]], rows[ri], gsem[ri])

        def wait_gather(j, ri, ii):
            pltpu.make_async_copy(x_hbm.at[idx[ii].at[0]], rows[ri],
                                  gsem[ri]).wait()

        def start_scatter(j, ri, ii):
            pltpu.async_copy(rows[ri], acc_sh.at[idx[ii].at[1]], ssem[ri],
                             add=True)

        def wait_scatter(j, ri, ii):
            pltpu.make_async_copy(rows[ri], acc_sh.at[idx[ii].at[1]],
                                  ssem[ri]).wait()

        def body(j, scatter_wait, idx_ahead, gather_ahead):
            r, i4 = j % 3, j % 4
            if scatter_wait:
                wait_scatter(j - 2, (j - 2) % 3, (j - 2) % 4)
            if gather_ahead:
                wait_idx(j + 1, (j + 1) % 4)
                start_gather(j + 1, (j + 1) % 3, (j + 1) % 4)
            if idx_ahead:
                start_idx(j + 2, (j + 2) % 4)
            wait_gather(j, r, i4)
            start_scatter(j, r, i4)

        start_idx(0, 0)
        start_idx(1, 1)
        wait_idx(0, 0)
        start_gather(0, 0, 0)
        body(0, scatter_wait=False, idx_ahead=True, gather_ahead=True)
        body(1, scatter_wait=False, idx_ahead=True, gather_ahead=True)

        def steady_step(g, _):
            for p in range(12):
                j12 = 12 * g + 2 + p
                wait_scatter(j12 - 2, p % 3, p % 4)
                wait_idx(j12 + 1, (p + 3) % 4)
                start_gather(j12 + 1, p % 3, (p + 3) % 4)
                start_idx(j12 + 2, p % 4)
                wait_gather(j12, (p + 2) % 3, (p + 2) % 4)
                start_scatter(j12, (p + 2) % 3, (p + 2) % 4)
            return 0

        nloop = (NCHUNK - 5) // 12
        lax.fori_loop(0, nloop, steady_step, 0)
        for j in range(12 * nloop + 2, NCHUNK):
            body(j, scatter_wait=True, idx_ahead=(j + 2 < NCHUNK),
                 gather_ahead=(j + 1 < NCHUNK))
        wait_scatter(NCHUNK - 2, (NCHUNK - 2) % 3, (NCHUNK - 2) % 4)
        wait_scatter(NCHUNK - 1, (NCHUNK - 1) % 3, (NCHUNK - 1) % 4)

    def flush(out_hbm):
        def f_step(i, _):
            off = pl.multiple_of(sbase + i * FR, 8)
            pltpu.sync_copy(acc_sh.at[pl.ds(off, FR)], rows0_v)
            pltpu.sync_copy(rows0_v, out_hbm.at[pl.ds(off, FR)])
            return 0

        lax.fori_loop(0, STRIPE // FR, f_step, 0)

    @pl.when(c == 0)
    def _():
        run(xl_hbm)

    @pl.when(c == 1)
    def _():
        run(xr_hbm)

    plsc.subcore_barrier()

    @pl.when(c == 0)
    def _():
        flush(outl_hbm)

    @pl.when(c == 1)
    def _():
        flush(outr_hbm)


def _edge_agg(xl, xr, e3):
    mesh = plsc.VectorSubcoreMesh(core_axis_name="c", subcore_axis_name="s",
                                  num_cores=NC, num_subcores=NS)
    return pl.kernel(
        _agg_body,
        out_type=(jax.ShapeDtypeStruct((N_PAD, DH), jnp.float32),
                  jax.ShapeDtypeStruct((N_PAD, DH), jnp.float32)),
        mesh=mesh,
        scratch_types=(
            [pltpu.VMEM((2, CK), jnp.int32)] * 4
            + [pltpu.VMEM((CK, DH), jnp.float32)] * 3
            + [pltpu.VMEM_SHARED((N_PAD, DH), jnp.float32)]
            + [pltpu.SemaphoreType.DMA] * 10
        ),
        compiler_params=pltpu.CompilerParams(needs_layout_passes=False),
    )(xl, xr, e3)


def _pool_body(aggl_ref, aggr_ref, xl_ref, xr_ref, degp_ref, b_ref, batch_ref,
               out_ref, sum_acc, cnt_acc):
    i = pl.program_id(0)
    deg = jnp.sum(degp_ref[0], axis=0) + 1.0
    dinv = lax.rsqrt(deg)
    agg = jnp.concatenate([aggl_ref[...], aggr_ref[...]], axis=1)
    xs = jnp.concatenate([xl_ref[...], xr_ref[...]], axis=1)
    h = jnp.maximum(dinv[:, None] * (agg + xs) + b_ref[...], 0.0)
    bvec = batch_ref[...].reshape(1, RB)
    onehot = (lax.broadcasted_iota(jnp.int32, (N_GRAPHS, RB), 0)
              == bvec).astype(jnp.float32)

    @pl.when(i == 0)
    def _():
        sum_acc[...] = jnp.zeros_like(sum_acc)
        cnt_acc[...] = jnp.zeros_like(cnt_acc)

    sum_acc[...] += jnp.dot(onehot, h, preferred_element_type=jnp.float32)
    cnt_acc[...] += jnp.dot(onehot, jnp.ones((RB, D_OUT), jnp.float32),
                            preferred_element_type=jnp.float32)

    @pl.when(i == NRB - 1)
    def _():
        out_ref[...] = sum_acc[...] / jnp.maximum(cnt_acc[...], 1.0)


def _pool(aggl, aggr, xl, xr, degp, b2, batch3):
    return pl.pallas_call(
        _pool_body,
        grid=(NRB,),
        in_specs=[
            pl.BlockSpec((RB, DH), lambda i: (i, 0)),
            pl.BlockSpec((RB, DH), lambda i: (i, 0)),
            pl.BlockSpec((RB, DH), lambda i: (i, 0)),
            pl.BlockSpec((RB, DH), lambda i: (i, 0)),
            pl.BlockSpec((1, NW, RB), lambda i: (i, 0, 0)),
            pl.BlockSpec((1, D_OUT), lambda i: (0, 0)),
            pl.BlockSpec((1, 1, RB), lambda i: (i, 0, 0)),
        ],
        out_specs=pl.BlockSpec((N_GRAPHS, D_OUT), lambda i: (0, 0)),
        out_shape=jax.ShapeDtypeStruct((N_GRAPHS, D_OUT), jnp.float32),
        scratch_shapes=[
            pltpu.VMEM((N_GRAPHS, D_OUT), jnp.float32),
            pltpu.VMEM((N_GRAPHS, D_OUT), jnp.float32),
        ],
    )(aggl, aggr, xl, xr, degp, b2, batch3)


def kernel(v, e, batch, W, b):
    e = e.astype(jnp.int32)
    row, col = e[0], e[1]
    pad = NW * EPW + PAD_E - N_EDGES
    col_padded = jnp.concatenate([col, jnp.zeros((pad,), jnp.int32)])
    degp = _deg_partials(col_padded)
    degp_t = degp.reshape(NRB, NW, RB)
    xl, xr = _scaled_x(v, W, degp_t)
    e3 = (e.reshape(2, NS, NCHUNK, CK).transpose(1, 2, 0, 3)
          .reshape(NS * NCHUNK, 2, CK))
    aggl, aggr = _edge_agg(xl, xr, e3)
    return _pool(aggl, aggr, xl, xr, degp_t, b.reshape(1, D_OUT),
                 batch.astype(jnp.int32).reshape(NRB, 1, RB))
